# baseline (speedup 1.0000x reference)
import functools

import jax
import jax.numpy as jnp
from jax import lax
from jax.experimental import pallas as pl
from jax.experimental.pallas import tpu as pltpu
from jax.experimental.pallas import tpu_sc as plsc

N = 10000
E = 320000
DIM = 128
L = 3
G = 16
GDIM = 128

NC = 2
NS = 16
NW = NC * NS
EPW = E // NW
K = 40
NPAD = 10240
RPT = NPAD // NS
ZR = 32
NZ = RPT // ZR
NCHUNK_P = 252
EPWP = NCHUNK_P * K

BLK = 1000
GRID = N // BLK


def _embed_body(x_ref, emb_ref, h_ref, stats_ref):
    i = pl.program_id(0)
    xv = x_ref[...]
    oh = (xv == lax.broadcasted_iota(jnp.int32, (BLK, 16), 1)).astype(jnp.float32)
    h = jnp.dot(oh, emb_ref[...], preferred_element_type=jnp.float32)
    h_ref[...] = h
    st = jnp.concatenate(
        [jnp.sum(h, axis=0, keepdims=True), jnp.sum(h * h, axis=0, keepdims=True)],
        axis=0)

    @pl.when(i == 0)
    def _():
        stats_ref[...] = st

    @pl.when(i > 0)
    def _():
        stats_ref[...] += st


_embed_call = pl.pallas_call(
    _embed_body,
    grid=(GRID,),
    in_specs=[
        pl.BlockSpec((BLK, 1), lambda i: (i, 0)),
        pl.BlockSpec((16, DIM), lambda i: (0, 0)),
    ],
    out_specs=[
        pl.BlockSpec((BLK, DIM), lambda i: (i, 0)),
        pl.BlockSpec((2, DIM), lambda i: (0, 0)),
    ],
    out_shape=[
        jax.ShapeDtypeStruct((N, DIM), jnp.float32),
        jax.ShapeDtypeStruct((2, DIM), jnp.float32),
    ],
)


def _dense_body(h_ref, stats_ref, aw, ab, bw, bb, vw, vb, uw, ub, gm, bt,
                ah_ref, bv_ref, hnu_ref, hn_ref):
    h = h_ref[...]
    mean = stats_ref[0:1, :] * (1.0 / N)
    ex2 = stats_ref[1:2, :] * (1.0 / N)
    var = ex2 - mean * mean
    scale = lax.rsqrt(var + 1e-5) * gm[...]
    hn = (h - mean) * scale + bt[...]
    hn_ref[...] = hn
    ah_ref[...] = jnp.dot(hn, aw[...], preferred_element_type=jnp.float32) + ab[...]
    bv_ref[:, 0:DIM] = jnp.dot(hn, bw[...], preferred_element_type=jnp.float32) + bb[...]
    bv_ref[:, DIM:2 * DIM] = jnp.dot(hn, vw[...], preferred_element_type=jnp.float32) + vb[...]
    hnu_ref[...] = jnp.dot(hn, uw[...], preferred_element_type=jnp.float32) + ub[...]


_w_spec = pl.BlockSpec((DIM, DIM), lambda i: (0, 0))
_b_spec = pl.BlockSpec((1, DIM), lambda i: (0, 0))

_dense_call = pl.pallas_call(
    _dense_body,
    grid=(GRID,),
    in_specs=[
        pl.BlockSpec((BLK, DIM), lambda i: (i, 0)),
        pl.BlockSpec((2, DIM), lambda i: (0, 0)),
        _w_spec, _b_spec, _w_spec, _b_spec, _w_spec, _b_spec, _w_spec, _b_spec,
        _b_spec, _b_spec,
    ],
    out_specs=[
        pl.BlockSpec((BLK, DIM), lambda i: (i, 0)),
        pl.BlockSpec((BLK, 2 * DIM), lambda i: (i, 0)),
        pl.BlockSpec((BLK, DIM), lambda i: (i, 0)),
        pl.BlockSpec((BLK, DIM), lambda i: (i, 0)),
    ],
    out_shape=[
        jax.ShapeDtypeStruct((N, DIM), jnp.float32),
        jax.ShapeDtypeStruct((N, 2 * DIM), jnp.float32),
        jax.ShapeDtypeStruct((N, DIM), jnp.float32),
        jax.ShapeDtypeStruct((N, DIM), jnp.float32),
    ],
)


def _combine_body(hnu_ref, aggr_ref, hn_ref, h_ref, stats_ref):
    i = pl.program_id(0)
    t = hnu_ref[...] + aggr_ref[0] + aggr_ref[1]
    h2 = jnp.where(t > 0, t, 0.01 * t)
    h = h2 + hn_ref[...]
    h_ref[...] = h
    st = jnp.concatenate(
        [jnp.sum(h, axis=0, keepdims=True), jnp.sum(h * h, axis=0, keepdims=True)],
        axis=0)

    @pl.when(i == 0)
    def _():
        stats_ref[...] = st

    @pl.when(i > 0)
    def _():
        stats_ref[...] += st


_combine_call = pl.pallas_call(
    _combine_body,
    grid=(GRID,),
    in_specs=[
        pl.BlockSpec((BLK, DIM), lambda i: (i, 0)),
        pl.BlockSpec((NC, BLK, DIM), lambda i: (0, i, 0)),
        pl.BlockSpec((BLK, DIM), lambda i: (i, 0)),
    ],
    out_specs=[
        pl.BlockSpec((BLK, DIM), lambda i: (i, 0)),
        pl.BlockSpec((2, DIM), lambda i: (0, 0)),
    ],
    out_shape=[
        jax.ShapeDtypeStruct((N, DIM), jnp.float32),
        jax.ShapeDtypeStruct((2, DIM), jnp.float32),
    ],
)


def _pool_body(h_ref, batch_ref, fw_ref, fb_ref, out_ref, acc_ref):
    i = pl.program_id(0)

    @pl.when(i == 0)
    def _():
        acc_ref[...] = jnp.zeros((G, DIM), jnp.float32)

    bv = batch_ref[...]
    oh = (bv == lax.broadcasted_iota(jnp.int32, (BLK, G), 1)).astype(jnp.float32)
    acc_ref[...] += lax.dot_general(
        oh, h_ref[...], (((0,), (0,)), ((), ())),
        preferred_element_type=jnp.float32)

    @pl.when(i == GRID - 1)
    def _():
        out_ref[...] = jnp.dot(acc_ref[...], fw_ref[...],
                               preferred_element_type=jnp.float32) + fb_ref[...]


_pool_call = pl.pallas_call(
    _pool_body,
    grid=(GRID,),
    in_specs=[
        pl.BlockSpec((BLK, DIM), lambda i: (i, 0)),
        pl.BlockSpec((BLK, 1), lambda i: (i, 0)),
        pl.BlockSpec((DIM, GDIM), lambda i: (0, 0)),
        pl.BlockSpec((1, GDIM), lambda i: (0, 0)),
    ],
    out_specs=pl.BlockSpec((G, GDIM), lambda i: (0, 0)),
    out_shape=jax.ShapeDtypeStruct((G, GDIM), jnp.float32),
    scratch_shapes=[pltpu.VMEM((G, DIM), jnp.float32)],
)


def _edge_body(ah_hbm, bv_hbm, dst_hbm, src_hbm, out_hbm,
               d0, d1, d2, d3, d4, d5, s0, s1, s2, s3, s4, s5,
               ahb0, ahb1, bvb0, bvb1, mb0, mb1, zb, aggr,
               semi0, semi1, semg0, semg1, semsc0, semsc1):
    cid = lax.axis_index("c")
    sid = lax.axis_index("s")
    wid = cid * NS + sid
    dsl = (d0, d1, d2, d3, d4, d5)
    ssl = (s0, s1, s2, s3, s4, s5)
    ahb = (ahb0, ahb1)
    bvb = (bvb0, bvb1)
    mb = (mb0, mb1)
    semi = (semi0, semi1)
    semg = (semg0, semg1)
    semsc = (semsc0, semsc1)

    def zrow(r, _):
        for c in range(DIM // 16):
            zb[r, pl.ds(c * 16, 16)] = jnp.zeros((16,), jnp.float32)
        return 0

    lax.fori_loop(0, ZR, zrow, 0)
    for j in range(NZ):
        pltpu.sync_copy(zb, aggr.at[pl.ds(sid * RPT + j * ZR, ZR)])
    plsc.subcore_barrier()

    base0 = wid * EPWP

    def idx_issue(i, slot):
        pltpu.async_copy(dst_hbm.at[pl.ds(base0 + i * K, K)], dsl[slot],
                         semi[slot % 2])
        pltpu.async_copy(src_hbm.at[pl.ds(base0 + i * K, K)], ssl[slot],
                         semi[slot % 2])

    def idx_drain(par):
        pltpu.make_async_copy(dst_hbm.at[pl.ds(0, K)], dsl[0], semi[par]).wait()
        pltpu.make_async_copy(dst_hbm.at[pl.ds(0, K)], ssl[0], semi[par]).wait()

    def gather_issue(slot6, b2):
        pltpu.async_copy(ah_hbm.at[dsl[slot6]], ahb[b2], semg[b2])
        pltpu.async_copy(bv_hbm.at[ssl[slot6]], bvb[b2], semg[b2])

    def gather_drain(b2):
        pltpu.make_async_copy(ah_hbm.at[pl.ds(0, K)], ahb[b2], semg[b2]).wait()
        pltpu.make_async_copy(bv_hbm.at[pl.ds(0, K)], bvb[b2], semg[b2]).wait()

    def scatter_drain(b2):
        pltpu.make_async_copy(mb[b2], aggr.at[pl.ds(0, K)], semsc[b2]).wait()

    idx_issue(0, 0)
    idx_issue(1, 1)
    idx_drain(0)
    gather_issue(0, 0)

    def super_chunk(s, _):
        for b in range(6):
            i = s * 6 + b
            p2, n2 = b % 2, (b + 1) % 2

            @pl.when(i >= 2)
            def _():
                scatter_drain(p2)

            @pl.when(i + 2 < NCHUNK_P)
            def _():
                idx_issue(i + 2, (b + 2) % 6)

            @pl.when(i + 1 < NCHUNK_P)
            def _():
                idx_drain(n2)
                gather_issue((b + 1) % 6, n2)

            gather_drain(p2)

            @plsc.parallel_loop(0, K, 1, unroll=6)
            def _(r):
                for cc in range(DIM // 16):
                    a = ahb[p2][r, pl.ds(cc * 16, 16)]
                    bh = bvb[p2][r, pl.ds(cc * 16, 16)]
                    v = bvb[p2][r, pl.ds(DIM + cc * 16, 16)]
                    mb[p2][r, pl.ds(cc * 16, 16)] = v / (1.0 + jnp.exp(-(a + bh)))

            pltpu.async_copy(mb[p2], aggr.at[dsl[b]], semsc[p2], add=True)
        return 0

    lax.fori_loop(0, NCHUNK_P // 6, super_chunk, 0)

    scatter_drain((NCHUNK_P - 2) % 2)
    scatter_drain((NCHUNK_P - 1) % 2)
    plsc.subcore_barrier()
    for j in range(NZ):
        r0 = sid * RPT + j * ZR
        pltpu.sync_copy(aggr.at[pl.ds(r0, ZR)], out_hbm.at[cid, pl.ds(r0, ZR)])


_edge_call = functools.partial(
    pl.kernel,
    out_type=jax.ShapeDtypeStruct((NC, NPAD, DIM), jnp.float32),
    mesh=plsc.VectorSubcoreMesh(
        core_axis_name="c", subcore_axis_name="s",
        num_cores=NC, num_subcores=NS),
    scratch_types=(
        [pltpu.VMEM((K,), jnp.int32)] * 12 +
        [pltpu.VMEM((K, DIM), jnp.float32)] * 2 +
        [pltpu.VMEM((K, 2 * DIM), jnp.float32)] * 2 +
        [pltpu.VMEM((K, DIM), jnp.float32)] * 2 +
        [pltpu.VMEM((ZR, DIM), jnp.float32)] +
        [pltpu.VMEM_SHARED((NPAD, DIM), jnp.float32)] +
        [pltpu.SemaphoreType.DMA] * 6
    ),
)(_edge_body)


def kernel(x, edge_index, batch, emb, Uw, Ub, Vw, Vb, Aw, Ab, Bw, Bb,
           gamma, beta, Fw, Fb):
    x2 = x.reshape(N, 1).astype(jnp.int32)
    batch2 = batch.reshape(N, 1).astype(jnp.int32)
    src = jnp.pad(edge_index[0].astype(jnp.int32).reshape(NW, EPW),
                  ((0, 0), (0, EPWP - EPW))).reshape(NW * EPWP)
    dst = jnp.pad(edge_index[1].astype(jnp.int32).reshape(NW, EPW),
                  ((0, 0), (0, EPWP - EPW)),
                  constant_values=N).reshape(NW * EPWP)

    h, stats = _embed_call(x2, emb)
    for l in range(L):
        ah, bv, hnu, hn = _dense_call(
            h, stats,
            Aw[l], Ab[l].reshape(1, DIM),
            Bw[l], Bb[l].reshape(1, DIM),
            Vw[l], Vb[l].reshape(1, DIM),
            Uw[l], Ub[l].reshape(1, DIM),
            gamma[l].reshape(1, DIM), beta[l].reshape(1, DIM))
        aggr = _edge_call(ah, bv, dst, src)
        h, stats = _combine_call(hnu, aggr, hn)
    out = _pool_call(h, batch2, Fw, Fb.reshape(1, GDIM))
    return out


# speedup vs baseline: 1.1956x; 1.1956x over previous
import functools

import jax
import jax.numpy as jnp
from jax import lax
from jax.experimental import pallas as pl
from jax.experimental.pallas import tpu as pltpu
from jax.experimental.pallas import tpu_sc as plsc

N = 10000
E = 320000
DIM = 128
L = 3
G = 16
GDIM = 128

NC = 2
NS = 16
NW = NC * NS
EPW = E // NW
K = 40
NPAD = 10240
RPT = NPAD // NS
ZR = 32
NZ = RPT // ZR
NCHUNK_P = 252
EPWP = NCHUNK_P * K

BLK = 1000
GRID = N // BLK


def _embed_body(x_ref, emb_ref, h_ref, stats_ref):
    i = pl.program_id(0)
    xv = x_ref[...]
    oh = (xv == lax.broadcasted_iota(jnp.int32, (BLK, 16), 1)).astype(jnp.float32)
    h = jnp.dot(oh, emb_ref[...], preferred_element_type=jnp.float32)
    h_ref[...] = h
    st = jnp.concatenate(
        [jnp.sum(h, axis=0, keepdims=True), jnp.sum(h * h, axis=0, keepdims=True)],
        axis=0)

    @pl.when(i == 0)
    def _():
        stats_ref[...] = st

    @pl.when(i > 0)
    def _():
        stats_ref[...] += st


_embed_call = pl.pallas_call(
    _embed_body,
    grid=(GRID,),
    in_specs=[
        pl.BlockSpec((BLK, 1), lambda i: (i, 0)),
        pl.BlockSpec((16, DIM), lambda i: (0, 0)),
    ],
    out_specs=[
        pl.BlockSpec((BLK, DIM), lambda i: (i, 0)),
        pl.BlockSpec((2, DIM), lambda i: (0, 0)),
    ],
    out_shape=[
        jax.ShapeDtypeStruct((N, DIM), jnp.float32),
        jax.ShapeDtypeStruct((2, DIM), jnp.float32),
    ],
)


def _dense_body(h_ref, stats_ref, aw, ab, bw, bb, vw, vb, uw, ub, gm, bt,
                ah_ref, bv_ref, hnu_ref, hn_ref):
    h = h_ref[...]
    mean = stats_ref[0:1, :] * (1.0 / N)
    ex2 = stats_ref[1:2, :] * (1.0 / N)
    var = ex2 - mean * mean
    scale = lax.rsqrt(var + 1e-5) * gm[...]
    hn = (h - mean) * scale + bt[...]
    hn_ref[...] = hn
    ah_ref[...] = jnp.dot(hn, aw[...], preferred_element_type=jnp.float32) + ab[...]
    bv_ref[:, 0:DIM] = jnp.dot(hn, bw[...], preferred_element_type=jnp.float32) + bb[...]
    bv_ref[:, DIM:2 * DIM] = jnp.dot(hn, vw[...], preferred_element_type=jnp.float32) + vb[...]
    hnu_ref[...] = jnp.dot(hn, uw[...], preferred_element_type=jnp.float32) + ub[...]


_w_spec = pl.BlockSpec((DIM, DIM), lambda i: (0, 0))
_b_spec = pl.BlockSpec((1, DIM), lambda i: (0, 0))

_dense_call = pl.pallas_call(
    _dense_body,
    grid=(GRID,),
    in_specs=[
        pl.BlockSpec((BLK, DIM), lambda i: (i, 0)),
        pl.BlockSpec((2, DIM), lambda i: (0, 0)),
        _w_spec, _b_spec, _w_spec, _b_spec, _w_spec, _b_spec, _w_spec, _b_spec,
        _b_spec, _b_spec,
    ],
    out_specs=[
        pl.BlockSpec((BLK, DIM), lambda i: (i, 0)),
        pl.BlockSpec((BLK, 2 * DIM), lambda i: (i, 0)),
        pl.BlockSpec((BLK, DIM), lambda i: (i, 0)),
        pl.BlockSpec((BLK, DIM), lambda i: (i, 0)),
    ],
    out_shape=[
        jax.ShapeDtypeStruct((N, DIM), jnp.float32),
        jax.ShapeDtypeStruct((N, 2 * DIM), jnp.float32),
        jax.ShapeDtypeStruct((N, DIM), jnp.float32),
        jax.ShapeDtypeStruct((N, DIM), jnp.float32),
    ],
)


def _combine_body(hnu_ref, aggr_ref, hn_ref, h_ref, stats_ref):
    i = pl.program_id(0)
    t = hnu_ref[...] + aggr_ref[0] + aggr_ref[1]
    h2 = jnp.where(t > 0, t, 0.01 * t)
    h = h2 + hn_ref[...]
    h_ref[...] = h
    st = jnp.concatenate(
        [jnp.sum(h, axis=0, keepdims=True), jnp.sum(h * h, axis=0, keepdims=True)],
        axis=0)

    @pl.when(i == 0)
    def _():
        stats_ref[...] = st

    @pl.when(i > 0)
    def _():
        stats_ref[...] += st


_combine_call = pl.pallas_call(
    _combine_body,
    grid=(GRID,),
    in_specs=[
        pl.BlockSpec((BLK, DIM), lambda i: (i, 0)),
        pl.BlockSpec((NC, BLK, DIM), lambda i: (0, i, 0)),
        pl.BlockSpec((BLK, DIM), lambda i: (i, 0)),
    ],
    out_specs=[
        pl.BlockSpec((BLK, DIM), lambda i: (i, 0)),
        pl.BlockSpec((2, DIM), lambda i: (0, 0)),
    ],
    out_shape=[
        jax.ShapeDtypeStruct((N, DIM), jnp.float32),
        jax.ShapeDtypeStruct((2, DIM), jnp.float32),
    ],
)


def _pool_body(h_ref, batch_ref, fw_ref, fb_ref, out_ref, acc_ref):
    i = pl.program_id(0)

    @pl.when(i == 0)
    def _():
        acc_ref[...] = jnp.zeros((G, DIM), jnp.float32)

    bv = batch_ref[...]
    oh = (bv == lax.broadcasted_iota(jnp.int32, (BLK, G), 1)).astype(jnp.float32)
    acc_ref[...] += lax.dot_general(
        oh, h_ref[...], (((0,), (0,)), ((), ())),
        preferred_element_type=jnp.float32)

    @pl.when(i == GRID - 1)
    def _():
        out_ref[...] = jnp.dot(acc_ref[...], fw_ref[...],
                               preferred_element_type=jnp.float32) + fb_ref[...]


_pool_call = pl.pallas_call(
    _pool_body,
    grid=(GRID,),
    in_specs=[
        pl.BlockSpec((BLK, DIM), lambda i: (i, 0)),
        pl.BlockSpec((BLK, 1), lambda i: (i, 0)),
        pl.BlockSpec((DIM, GDIM), lambda i: (0, 0)),
        pl.BlockSpec((1, GDIM), lambda i: (0, 0)),
    ],
    out_specs=pl.BlockSpec((G, GDIM), lambda i: (0, 0)),
    out_shape=jax.ShapeDtypeStruct((G, GDIM), jnp.float32),
    scratch_shapes=[pltpu.VMEM((G, DIM), jnp.float32)],
)


def _edge_body(ah_hbm, bv_hbm, dst_hbm, src_hbm, out_hbm,
               d0, d1, d2, d3, d4, d5, s0, s1, s2, s3, s4, s5,
               ahb0, ahb1, bvb0, bvb1, mb0, mb1, zb, aggr,
               semi0, semi1, semg0, semg1, semsc0, semsc1):
    cid = lax.axis_index("c")
    sid = lax.axis_index("s")
    wid = cid * NS + sid
    dsl = (d0, d1, d2, d3, d4, d5)
    ssl = (s0, s1, s2, s3, s4, s5)
    ahb = (ahb0, ahb1)
    bvb = (bvb0, bvb1)
    mb = (mb0, mb1)
    semi = (semi0, semi1)
    semg = (semg0, semg1)
    semsc = (semsc0, semsc1)

    def zrow(r, _):
        for c in range(DIM // 16):
            zb[r, pl.ds(c * 16, 16)] = jnp.zeros((16,), jnp.float32)
        return 0

    lax.fori_loop(0, ZR, zrow, 0)
    for j in range(NZ):
        pltpu.sync_copy(zb, aggr.at[pl.ds(sid * RPT + j * ZR, ZR)])
    plsc.subcore_barrier()

    base0 = wid * EPWP

    def idx_issue(i, slot):
        pltpu.async_copy(dst_hbm.at[pl.ds(base0 + i * K, K)], dsl[slot],
                         semi[slot % 2])
        pltpu.async_copy(src_hbm.at[pl.ds(base0 + i * K, K)], ssl[slot],
                         semi[slot % 2])

    def idx_drain(par):
        pltpu.make_async_copy(dst_hbm.at[pl.ds(0, K)], dsl[0], semi[par]).wait()
        pltpu.make_async_copy(dst_hbm.at[pl.ds(0, K)], ssl[0], semi[par]).wait()

    def gather_issue(slot6, b2):
        pltpu.async_copy(ah_hbm.at[dsl[slot6]], ahb[b2], semg[b2])
        pltpu.async_copy(bv_hbm.at[ssl[slot6]], bvb[b2], semg[b2])

    def gather_drain(b2):
        pltpu.make_async_copy(ah_hbm.at[pl.ds(0, K)], ahb[b2], semg[b2]).wait()
        pltpu.make_async_copy(bv_hbm.at[pl.ds(0, K)], bvb[b2], semg[b2]).wait()

    def scatter_drain(b2):
        pltpu.make_async_copy(mb[b2], aggr.at[pl.ds(0, K)], semsc[b2]).wait()

    idx_issue(0, 0)
    idx_issue(1, 1)
    idx_drain(0)
    gather_issue(0, 0)

    def super_chunk(s, _):
        for b in range(6):
            i = s * 6 + b
            p2, n2 = b % 2, (b + 1) % 2

            @pl.when(i >= 2)
            def _():
                scatter_drain(p2)

            @pl.when(i + 2 < NCHUNK_P)
            def _():
                idx_issue(i + 2, (b + 2) % 6)

            @pl.when(i + 1 < NCHUNK_P)
            def _():
                idx_drain(n2)
                gather_issue((b + 1) % 6, n2)

            gather_drain(p2)

            @plsc.parallel_loop(0, K, 1, unroll=4)
            def _(r):
                for cc in range(DIM // 16):
                    a = ahb[p2][r, pl.ds(cc * 16, 16)]
                    bh = bvb[p2][r, pl.ds(cc * 16, 16)]
                    v = bvb[p2][r, pl.ds(DIM + cc * 16, 16)]
                    mb[p2][r, pl.ds(cc * 16, 16)] = v / (1.0 + jnp.exp(-(a + bh)))

            pltpu.async_copy(mb[p2], aggr.at[dsl[b]], semsc[p2], add=True)
        return 0

    lax.fori_loop(0, NCHUNK_P // 6, super_chunk, 0)

    scatter_drain((NCHUNK_P - 2) % 2)
    scatter_drain((NCHUNK_P - 1) % 2)
    plsc.subcore_barrier()
    for j in range(NZ):
        r0 = sid * RPT + j * ZR
        pltpu.sync_copy(aggr.at[pl.ds(r0, ZR)], out_hbm.at[cid, pl.ds(r0, ZR)])


_edge_call = functools.partial(
    pl.kernel,
    out_type=jax.ShapeDtypeStruct((NC, NPAD, DIM), jnp.float32),
    mesh=plsc.VectorSubcoreMesh(
        core_axis_name="c", subcore_axis_name="s",
        num_cores=NC, num_subcores=NS),
    scratch_types=(
        [pltpu.VMEM((K,), jnp.int32)] * 12 +
        [pltpu.VMEM((K, DIM), jnp.float32)] * 2 +
        [pltpu.VMEM((K, 2 * DIM), jnp.float32)] * 2 +
        [pltpu.VMEM((K, DIM), jnp.float32)] * 2 +
        [pltpu.VMEM((ZR, DIM), jnp.float32)] +
        [pltpu.VMEM_SHARED((NPAD, DIM), jnp.float32)] +
        [pltpu.SemaphoreType.DMA] * 6
    ),
)(_edge_body)


def kernel(x, edge_index, batch, emb, Uw, Ub, Vw, Vb, Aw, Ab, Bw, Bb,
           gamma, beta, Fw, Fb):
    x2 = x.reshape(N, 1).astype(jnp.int32)
    batch2 = batch.reshape(N, 1).astype(jnp.int32)
    src = jnp.pad(edge_index[0].astype(jnp.int32).reshape(NW, EPW),
                  ((0, 0), (0, EPWP - EPW))).reshape(NW * EPWP)
    dst = jnp.pad(edge_index[1].astype(jnp.int32).reshape(NW, EPW),
                  ((0, 0), (0, EPWP - EPW)),
                  constant_values=N).reshape(NW * EPWP)

    h, stats = _embed_call(x2, emb)
    for l in range(L):
        ah, bv, hnu, hn = _dense_call(
            h, stats,
            Aw[l], Ab[l].reshape(1, DIM),
            Bw[l], Bb[l].reshape(1, DIM),
            Vw[l], Vb[l].reshape(1, DIM),
            Uw[l], Ub[l].reshape(1, DIM),
            gamma[l].reshape(1, DIM), beta[l].reshape(1, DIM))
        aggr = _edge_call(ah, bv, dst, src)
        h, stats = _combine_call(hnu, aggr, hn)
    out = _pool_call(h, batch2, Fw, Fb.reshape(1, GDIM))
    return out


# speedup vs baseline: 1.3126x; 1.0979x over previous
import functools

import jax
import jax.numpy as jnp
from jax import lax
from jax.experimental import pallas as pl
from jax.experimental.pallas import tpu as pltpu
from jax.experimental.pallas import tpu_sc as plsc

N = 10000
E = 320000
DIM = 128
L = 3
G = 16
GDIM = 128

NC = 2
NS = 16
NW = NC * NS
EPW = E // NW
K = 40
NPAD = 10240
RPT = NPAD // NS
ZR = 32
NZ = RPT // ZR
NCHUNK_P = 252
EPWP = NCHUNK_P * K

BLK = 1000
GRID = N // BLK


def _embed_body(x_ref, emb_ref, h_ref, stats_ref):
    i = pl.program_id(0)
    xv = x_ref[...]
    oh = (xv == lax.broadcasted_iota(jnp.int32, (BLK, 16), 1)).astype(jnp.float32)
    h = jnp.dot(oh, emb_ref[...], preferred_element_type=jnp.float32)
    h_ref[...] = h
    st = jnp.concatenate(
        [jnp.sum(h, axis=0, keepdims=True), jnp.sum(h * h, axis=0, keepdims=True)],
        axis=0)

    @pl.when(i == 0)
    def _():
        stats_ref[...] = st

    @pl.when(i > 0)
    def _():
        stats_ref[...] += st


_embed_call = pl.pallas_call(
    _embed_body,
    grid=(GRID,),
    in_specs=[
        pl.BlockSpec((BLK, 1), lambda i: (i, 0)),
        pl.BlockSpec((16, DIM), lambda i: (0, 0)),
    ],
    out_specs=[
        pl.BlockSpec((BLK, DIM), lambda i: (i, 0)),
        pl.BlockSpec((2, DIM), lambda i: (0, 0)),
    ],
    out_shape=[
        jax.ShapeDtypeStruct((N, DIM), jnp.float32),
        jax.ShapeDtypeStruct((2, DIM), jnp.float32),
    ],
)


def _dense_body(h_ref, stats_ref, aw, ab, bw, bb, vw, vb, uw, ub, gm, bt,
                ah_ref, bv_ref, hnu_ref, hn_ref):
    h = h_ref[...]
    mean = stats_ref[0:1, :] * (1.0 / N)
    ex2 = stats_ref[1:2, :] * (1.0 / N)
    var = ex2 - mean * mean
    scale = lax.rsqrt(var + 1e-5) * gm[...]
    hn = (h - mean) * scale + bt[...]
    hn_ref[...] = hn
    ah_ref[...] = jnp.dot(hn, aw[...], preferred_element_type=jnp.float32) + ab[...]
    bv_ref[:, 0:DIM] = jnp.dot(hn, bw[...], preferred_element_type=jnp.float32) + bb[...]
    bv_ref[:, DIM:2 * DIM] = jnp.dot(hn, vw[...], preferred_element_type=jnp.float32) + vb[...]
    hnu_ref[...] = jnp.dot(hn, uw[...], preferred_element_type=jnp.float32) + ub[...]


_w_spec = pl.BlockSpec((DIM, DIM), lambda i: (0, 0))
_b_spec = pl.BlockSpec((1, DIM), lambda i: (0, 0))

_dense_call = pl.pallas_call(
    _dense_body,
    grid=(GRID,),
    in_specs=[
        pl.BlockSpec((BLK, DIM), lambda i: (i, 0)),
        pl.BlockSpec((2, DIM), lambda i: (0, 0)),
        _w_spec, _b_spec, _w_spec, _b_spec, _w_spec, _b_spec, _w_spec, _b_spec,
        _b_spec, _b_spec,
    ],
    out_specs=[
        pl.BlockSpec((BLK, DIM), lambda i: (i, 0)),
        pl.BlockSpec((BLK, 2 * DIM), lambda i: (i, 0)),
        pl.BlockSpec((BLK, DIM), lambda i: (i, 0)),
        pl.BlockSpec((BLK, DIM), lambda i: (i, 0)),
    ],
    out_shape=[
        jax.ShapeDtypeStruct((N, DIM), jnp.float32),
        jax.ShapeDtypeStruct((N, 2 * DIM), jnp.float32),
        jax.ShapeDtypeStruct((N, DIM), jnp.float32),
        jax.ShapeDtypeStruct((N, DIM), jnp.float32),
    ],
)


def _combine_body(hnu_ref, aggr_ref, hn_ref, h_ref, stats_ref):
    i = pl.program_id(0)
    t = hnu_ref[...] + aggr_ref[0] + aggr_ref[1]
    h2 = jnp.where(t > 0, t, 0.01 * t)
    h = h2 + hn_ref[...]
    h_ref[...] = h
    st = jnp.concatenate(
        [jnp.sum(h, axis=0, keepdims=True), jnp.sum(h * h, axis=0, keepdims=True)],
        axis=0)

    @pl.when(i == 0)
    def _():
        stats_ref[...] = st

    @pl.when(i > 0)
    def _():
        stats_ref[...] += st


_combine_call = pl.pallas_call(
    _combine_body,
    grid=(GRID,),
    in_specs=[
        pl.BlockSpec((BLK, DIM), lambda i: (i, 0)),
        pl.BlockSpec((NC, BLK, DIM), lambda i: (0, i, 0)),
        pl.BlockSpec((BLK, DIM), lambda i: (i, 0)),
    ],
    out_specs=[
        pl.BlockSpec((BLK, DIM), lambda i: (i, 0)),
        pl.BlockSpec((2, DIM), lambda i: (0, 0)),
    ],
    out_shape=[
        jax.ShapeDtypeStruct((N, DIM), jnp.float32),
        jax.ShapeDtypeStruct((2, DIM), jnp.float32),
    ],
)


def _pool_body(h_ref, batch_ref, fw_ref, fb_ref, out_ref, acc_ref):
    i = pl.program_id(0)

    @pl.when(i == 0)
    def _():
        acc_ref[...] = jnp.zeros((G, DIM), jnp.float32)

    bv = batch_ref[...]
    oh = (bv == lax.broadcasted_iota(jnp.int32, (BLK, G), 1)).astype(jnp.float32)
    acc_ref[...] += lax.dot_general(
        oh, h_ref[...], (((0,), (0,)), ((), ())),
        preferred_element_type=jnp.float32)

    @pl.when(i == GRID - 1)
    def _():
        out_ref[...] = jnp.dot(acc_ref[...], fw_ref[...],
                               preferred_element_type=jnp.float32) + fb_ref[...]


_pool_call = pl.pallas_call(
    _pool_body,
    grid=(GRID,),
    in_specs=[
        pl.BlockSpec((BLK, DIM), lambda i: (i, 0)),
        pl.BlockSpec((BLK, 1), lambda i: (i, 0)),
        pl.BlockSpec((DIM, GDIM), lambda i: (0, 0)),
        pl.BlockSpec((1, GDIM), lambda i: (0, 0)),
    ],
    out_specs=pl.BlockSpec((G, GDIM), lambda i: (0, 0)),
    out_shape=jax.ShapeDtypeStruct((G, GDIM), jnp.float32),
    scratch_shapes=[pltpu.VMEM((G, DIM), jnp.float32)],
)


def _edge_body(ah_hbm, bv_hbm, dst_hbm, src_hbm, out_hbm,
               d0, d1, d2, d3, d4, d5, s0, s1, s2, s3, s4, s5,
               ahb0, ahb1, bvb0, bvb1, mb0, mb1, zb, aggr,
               semi0, semi1, semg0, semg1, semsc0, semsc1):
    cid = lax.axis_index("c")
    sid = lax.axis_index("s")
    wid = cid * NS + sid
    dsl = (d0, d1, d2, d3, d4, d5)
    ssl = (s0, s1, s2, s3, s4, s5)
    ahb = (ahb0, ahb1)
    bvb = (bvb0, bvb1)
    mb = (mb0, mb1)
    semi = (semi0, semi1)
    semg = (semg0, semg1)
    semsc = (semsc0, semsc1)

    def zrow(r, _):
        for c in range(DIM // 16):
            zb[r, pl.ds(c * 16, 16)] = jnp.zeros((16,), jnp.float32)
        return 0

    lax.fori_loop(0, ZR, zrow, 0)
    for j in range(NZ):
        pltpu.sync_copy(zb, aggr.at[pl.ds(sid * RPT + j * ZR, ZR)])
    plsc.subcore_barrier()

    base0 = wid * EPWP

    def idx_issue(i, slot):
        pltpu.async_copy(dst_hbm.at[pl.ds(base0 + i * K, K)], dsl[slot],
                         semi[slot % 2])
        pltpu.async_copy(src_hbm.at[pl.ds(base0 + i * K, K)], ssl[slot],
                         semi[slot % 2])

    def idx_drain(par):
        pltpu.make_async_copy(dst_hbm.at[pl.ds(0, K)], dsl[0], semi[par]).wait()
        pltpu.make_async_copy(dst_hbm.at[pl.ds(0, K)], ssl[0], semi[par]).wait()

    def gather_issue(slot6, b2):
        pltpu.async_copy(ah_hbm.at[dsl[slot6]], ahb[b2], semg[b2])
        pltpu.async_copy(bv_hbm.at[ssl[slot6]], bvb[b2], semg[b2])

    def gather_drain(b2):
        pltpu.make_async_copy(ah_hbm.at[pl.ds(0, K)], ahb[b2], semg[b2]).wait()
        pltpu.make_async_copy(bv_hbm.at[pl.ds(0, K)], bvb[b2], semg[b2]).wait()

    def scatter_drain(b2):
        pltpu.make_async_copy(mb[b2], aggr.at[pl.ds(0, K)], semsc[b2]).wait()

    idx_issue(0, 0)
    idx_issue(1, 1)
    idx_drain(0)
    gather_issue(0, 0)

    def super_chunk(s, _):
        for b in range(6):
            i = s * 6 + b
            p2, n2 = b % 2, (b + 1) % 2

            @pl.when(i >= 2)
            def _():
                scatter_drain(p2)

            @pl.when(i + 2 < NCHUNK_P)
            def _():
                idx_issue(i + 2, (b + 2) % 6)

            @pl.when(i + 1 < NCHUNK_P)
            def _():
                idx_drain(n2)
                gather_issue((b + 1) % 6, n2)

            gather_drain(p2)

            @plsc.parallel_loop(0, K, 1, unroll=5)
            def _(r):
                for cc in range(DIM // 16):
                    a = ahb[p2][r, pl.ds(cc * 16, 16)]
                    bh = bvb[p2][r, pl.ds(cc * 16, 16)]
                    v = bvb[p2][r, pl.ds(DIM + cc * 16, 16)]
                    mb[p2][r, pl.ds(cc * 16, 16)] = v / (1.0 + jnp.exp(-(a + bh)))

            pltpu.async_copy(mb[p2], aggr.at[dsl[b]], semsc[p2], add=True)
        return 0

    lax.fori_loop(0, NCHUNK_P // 6, super_chunk, 0)

    scatter_drain((NCHUNK_P - 2) % 2)
    scatter_drain((NCHUNK_P - 1) % 2)
    plsc.subcore_barrier()
    for j in range(NZ):
        r0 = sid * RPT + j * ZR
        pltpu.sync_copy(aggr.at[pl.ds(r0, ZR)], out_hbm.at[cid, pl.ds(r0, ZR)])


_edge_call = functools.partial(
    pl.kernel,
    out_type=jax.ShapeDtypeStruct((NC, NPAD, DIM), jnp.float32),
    mesh=plsc.VectorSubcoreMesh(
        core_axis_name="c", subcore_axis_name="s",
        num_cores=NC, num_subcores=NS),
    scratch_types=(
        [pltpu.VMEM((K,), jnp.int32)] * 12 +
        [pltpu.VMEM((K, DIM), jnp.float32)] * 2 +
        [pltpu.VMEM((K, 2 * DIM), jnp.float32)] * 2 +
        [pltpu.VMEM((K, DIM), jnp.float32)] * 2 +
        [pltpu.VMEM((ZR, DIM), jnp.float32)] +
        [pltpu.VMEM_SHARED((NPAD, DIM), jnp.float32)] +
        [pltpu.SemaphoreType.DMA] * 6
    ),
)(_edge_body)


def kernel(x, edge_index, batch, emb, Uw, Ub, Vw, Vb, Aw, Ab, Bw, Bb,
           gamma, beta, Fw, Fb):
    x2 = x.reshape(N, 1).astype(jnp.int32)
    batch2 = batch.reshape(N, 1).astype(jnp.int32)
    src = jnp.pad(edge_index[0].astype(jnp.int32).reshape(NW, EPW),
                  ((0, 0), (0, EPWP - EPW))).reshape(NW * EPWP)
    dst = jnp.pad(edge_index[1].astype(jnp.int32).reshape(NW, EPW),
                  ((0, 0), (0, EPWP - EPW)),
                  constant_values=N).reshape(NW * EPWP)

    h, stats = _embed_call(x2, emb)
    for l in range(L):
        ah, bv, hnu, hn = _dense_call(
            h, stats,
            Aw[l], Ab[l].reshape(1, DIM),
            Bw[l], Bb[l].reshape(1, DIM),
            Vw[l], Vb[l].reshape(1, DIM),
            Uw[l], Ub[l].reshape(1, DIM),
            gamma[l].reshape(1, DIM), beta[l].reshape(1, DIM))
        aggr = _edge_call(ah, bv, dst, src)
        h, stats = _combine_call(hnu, aggr, hn)
    out = _pool_call(h, batch2, Fw, Fb.reshape(1, GDIM))
    return out
